# baseline (speedup 1.0000x reference)
import functools

import jax
import jax.numpy as jnp
from jax import lax
from jax.experimental import pallas as pl
from jax.experimental.pallas import tpu as pltpu
from jax.experimental.pallas import tpu_sc as plsc

_DIM = 32
_WD = 0.01
_NC = 2
_NS = 16
_NW = _NC * _NS
_LANES = 16
_CHUNK = 128


def _sc_body(nchunks, u_hbm, i_hbm, j_hbm, W_hbm, H_hbm, x_hbm, reg_hbm,
             idx_u, idx_i, idx_j, u_rows, i_rows, j_rows, x_v, reg_v, sem):
    wid = lax.axis_index("s") * _NC + lax.axis_index("c")
    bpw = nchunks * _CHUNK
    base = wid * bpw

    cps = [
        pltpu.async_copy(u_hbm.at[wid], idx_u, sem),
        pltpu.async_copy(i_hbm.at[wid], idx_i, sem),
        pltpu.async_copy(j_hbm.at[wid], idx_j, sem),
    ]
    for cp in cps:
        cp.wait()

    cps = []
    for k in range(nchunks):
        sl = pl.ds(k * _CHUNK, _CHUNK)
        cps.append(pltpu.async_copy(W_hbm.at[idx_u.at[k]], u_rows.at[sl], sem))
        cps.append(pltpu.async_copy(H_hbm.at[idx_i.at[k]], i_rows.at[sl], sem))
        cps.append(pltpu.async_copy(H_hbm.at[idx_j.at[k]], j_rows.at[sl], sem))
    for cp in cps:
        cp.wait()

    lane = lax.broadcasted_iota(jnp.int32, (_LANES,), 0)

    def group(g, reg_acc):
        row_idx = lane + g * _LANES
        acc_ui = jnp.zeros((_LANES,), jnp.float32)
        acc_uj = jnp.zeros((_LANES,), jnp.float32)
        sq = reg_acc
        for d in range(_DIM):
            col = jnp.full((_LANES,), d, jnp.int32)
            uc = plsc.load_gather(u_rows, [row_idx, col])
            ic = plsc.load_gather(i_rows, [row_idx, col])
            jc = plsc.load_gather(j_rows, [row_idx, col])
            acc_ui = acc_ui + uc * ic
            acc_uj = acc_uj + uc * jc
            sq = sq + (uc * uc + ic * ic + jc * jc)
        x_v[pl.ds(g * _LANES, _LANES)] = acc_ui - acc_uj
        return sq

    ngroups = bpw // _LANES
    reg = lax.fori_loop(0, ngroups, group, jnp.zeros((_LANES,), jnp.float32))
    reg_v[...] = reg
    pltpu.sync_copy(x_v, x_hbm.at[pl.ds(base, bpw)])
    pltpu.sync_copy(reg_v, reg_hbm.at[wid])


def _tc_body(x_ref, reg_ref, out_ref):
    xs = x_ref[...]
    ls = jnp.minimum(xs, 0.0) - jnp.log1p(jnp.exp(-jnp.abs(xs)))
    out_ref[0, 0] = -jnp.sum(ls) + _WD * jnp.sum(reg_ref[...])


def kernel(u, i, j, W, H):
    B = u.shape[0]
    nchunks = B // (_NW * _CHUNK)
    mesh = plsc.VectorSubcoreMesh(core_axis_name="c", subcore_axis_name="s")
    bpw = nchunks * _CHUNK

    sc = pl.kernel(
        functools.partial(_sc_body, nchunks),
        out_type=(
            jax.ShapeDtypeStruct((B,), jnp.float32),
            jax.ShapeDtypeStruct((_NW, _LANES), jnp.float32),
        ),
        mesh=mesh,
        compiler_params=pltpu.CompilerParams(
            needs_layout_passes=False, use_tc_tiling_on_sc=False),
        scratch_types=[
            pltpu.VMEM((nchunks, _CHUNK), jnp.int32),
            pltpu.VMEM((nchunks, _CHUNK), jnp.int32),
            pltpu.VMEM((nchunks, _CHUNK), jnp.int32),
            pltpu.VMEM((bpw, _DIM), jnp.float32),
            pltpu.VMEM((bpw, _DIM), jnp.float32),
            pltpu.VMEM((bpw, _DIM), jnp.float32),
            pltpu.VMEM((bpw,), jnp.float32),
            pltpu.VMEM((_LANES,), jnp.float32),
            pltpu.SemaphoreType.DMA,
        ],
    )

    u3 = u.astype(jnp.int32).reshape(_NW, nchunks, _CHUNK)
    i3 = i.astype(jnp.int32).reshape(_NW, nchunks, _CHUNK)
    j3 = j.astype(jnp.int32).reshape(_NW, nchunks, _CHUNK)
    x, reg = sc(u3, i3, j3, W, H)

    out = pl.pallas_call(
        _tc_body,
        out_shape=jax.ShapeDtypeStruct((1, 1), jnp.float32),
        out_specs=pl.BlockSpec(memory_space=pltpu.SMEM),
    )(x.reshape(B // 128, 128), reg)
    return out.reshape(())
